# baseline (speedup 1.0000x reference)
import functools

import jax
import jax.numpy as jnp
from jax.experimental import pallas as pl
from jax.experimental.pallas import tpu as pltpu

NC = 200
D = 128
MM = 50
MP = 64
NS = 4000
LL = 200
BT = 64
BP = 32
NROWS = LL * BP
CH = 800
NCHUNK = NROWS // CH
TCH = CH // BP
GBLK = 400


def _hgnn_body(g_ref, x_ref, w_ref, b_ref, o_ref, *, relu):
    xw = jnp.dot(x_ref[...], w_ref[...],
                 preferred_element_type=jnp.float32) + b_ref[...]
    y = jnp.dot(g_ref[...], xw, preferred_element_type=jnp.float32)
    o_ref[...] = jnp.maximum(y, 0.0) if relu else y


def _hgnn_layer(G, X, W, b2d, relu):
    return pl.pallas_call(
        functools.partial(_hgnn_body, relu=relu),
        grid=(NS // GBLK,),
        in_specs=[
            pl.BlockSpec((GBLK, NS), lambda i: (i, 0)),
            pl.BlockSpec((NS, D), lambda i: (0, 0)),
            pl.BlockSpec((D, D), lambda i: (0, 0)),
            pl.BlockSpec((1, D), lambda i: (0, 0)),
        ],
        out_specs=pl.BlockSpec((GBLK, D), lambda i: (i, 0)),
        out_shape=jax.ShapeDtypeStruct((NS, D), jnp.float32),
        compiler_params=pltpu.CompilerParams(
            dimension_semantics=("parallel",)),
    )(G, X, W, b2d)


def _onehot_gather(idx_col, table_val, width):
    iota = jax.lax.broadcasted_iota(jnp.int32, (idx_col.shape[0], width), 1)
    oh = jnp.where(iota == idx_col, 1.0, 0.0).astype(jnp.float32)
    return jnp.dot(oh, table_val, preferred_element_type=jnp.float32)


def _main_body(skF_ref, anF_ref, stF_ref, semb_ref, kemb_ref, vemb_ref,
               mkT_ref, mv0_ref, wgk_ref, bgk_ref, wgv_ref, bgv_ref,
               we_ref, be_ref, wa_ref, ba_ref, wf_ref, bf_ref,
               wp_ref, bp_ref,
               out_ref, k_scr, w_scr, e_scr, a_scr, r_scr, mv_scr):
    wgk_s = wgk_ref[:D, :]
    wgk_k = wgk_ref[D:, :]
    wgv_s = wgv_ref[:D, :]
    wgv_k = wgv_ref[D:, :]
    wf_r = wf_ref[:D, :]
    wf_k = wf_ref[D:, :]

    for c in range(NCHUNK):
        sk = skF_ref[0, pl.ds(c * CH, CH), :]
        an = anF_ref[0, pl.ds(c * CH, CH), :]
        st = stF_ref[0, pl.ds(c * CH, CH), :] - 1
        ax = jnp.where(an == 2, 1, an)
        xv = sk + NC * ax

        sh0 = _onehot_gather(st[:CH // 2], semb_ref[...], NS)
        sh1 = _onehot_gather(st[CH // 2:], semb_ref[...], NS)
        shc = jnp.concatenate([sh0, sh1], axis=0)

        k0 = _onehot_gather(sk, kemb_ref[...], 208)
        v0 = _onehot_gather(xv, vemb_ref[...], 408)

        gk = jax.nn.sigmoid(
            jnp.dot(shc, wgk_s, preferred_element_type=jnp.float32)
            + jnp.dot(k0, wgk_k, preferred_element_type=jnp.float32)
            + bgk_ref[...])
        kc = gk * shc + (1.0 - gk) * k0
        gv = jax.nn.sigmoid(
            jnp.dot(shc, wgv_s, preferred_element_type=jnp.float32)
            + jnp.dot(v0, wgv_k, preferred_element_type=jnp.float32)
            + bgv_ref[...])
        vc = gv * shc + (1.0 - gv) * v0

        logits = jnp.dot(kc, mkT_ref[...], preferred_element_type=jnp.float32)
        miota = jax.lax.broadcasted_iota(jnp.int32, (CH, MP), 1)
        logits = jnp.where(miota < MM, logits, -1e30)
        logits = logits - jnp.max(logits, axis=1, keepdims=True)
        ex = jnp.exp(logits)
        wc = ex / jnp.sum(ex, axis=1, keepdims=True)

        ec = jax.nn.sigmoid(
            jnp.dot(vc, we_ref[...], preferred_element_type=jnp.float32)
            + be_ref[...])
        ac = jnp.tanh(
            jnp.dot(vc, wa_ref[...], preferred_element_type=jnp.float32)
            + ba_ref[...])

        k_scr[pl.ds(c * CH, CH), :] = kc
        w_scr[pl.ds(c * CH, CH), :] = wc
        e_scr[pl.ds(c * CH, CH), :] = ec
        a_scr[pl.ds(c * CH, CH), :] = ac

    mv_scr[...] = jnp.broadcast_to(mv0_ref[...][None], (BP, MP, D))

    li = jax.lax.broadcasted_iota(jnp.int32, (BP, BP * MP), 1) // MP
    bi = jax.lax.broadcasted_iota(jnp.int32, (BP, BP * MP), 0)
    bd_mask = li == bi

    def step(t, carry):
        wt = w_scr[pl.ds(t * BP, BP), :]
        et = e_scr[pl.ds(t * BP, BP), :]
        at = a_scr[pl.ds(t * BP, BP), :]
        mv = mv_scr[...]
        wblk = jnp.where(bd_mask, pltpu.repeat(wt, BP, axis=1), 0.0)
        r_scr[pl.ds(t * BP, BP), :] = jnp.dot(
            wblk, mv.reshape(BP * MP, D),
            preferred_element_type=jnp.float32)
        wt3 = wt[:, :, None]
        mv_scr[...] = mv - wt3 * (et[:, None, :] * mv - at[:, None, :])
        return carry

    jax.lax.fori_loop(0, LL, step, 0)

    for c in range(NCHUNK):
        rc = r_scr[pl.ds(c * CH, CH), :]
        kc = k_scr[pl.ds(c * CH, CH), :]
        fc = jnp.tanh(
            jnp.dot(rc, wf_r, preferred_element_type=jnp.float32)
            + jnp.dot(kc, wf_k, preferred_element_type=jnp.float32)
            + bf_ref[...])
        pc = jax.nn.sigmoid(
            jnp.dot(fc, wp_ref[...], preferred_element_type=jnp.float32)
            + bp_ref[...])
        nsk = skF_ref[0, pl.ds(c * CH + BP, CH), :]
        piota = jax.lax.broadcasted_iota(jnp.int32, (CH, NC), 1)
        pred = jnp.sum(jnp.where(piota == nsk, pc, 0.0), axis=1,
                       keepdims=True)
        out_ref[0, pl.ds(c * CH, CH), :] = pred


def kernel(student, skill, answer, G, stu_table, W1, b1, W2, b2, k_emb,
           v_emb, Mk, Mv0, Wgk, bgk, Wgv, bgv, We, be, Wa, ba, Wf, bf,
           Wp, bp):
    h = _hgnn_layer(G, stu_table, W1, b1.reshape(1, D), True)
    stu_emb = _hgnn_layer(G, h, W2, b2.reshape(1, D), False)

    def prep(x, pad_rows):
        xt = x.T.astype(jnp.int32).reshape(LL, 2, BP).transpose(1, 0, 2)
        xt = xt.reshape(2, NROWS, 1)
        return jnp.pad(xt, ((0, 0), (0, pad_rows), (0, 0)))
    skF = prep(skill, BP)
    anF = prep(answer, 0)
    stF = prep(student, 0)

    kemb_p = jnp.pad(k_emb, ((0, 208 - (NC + 1)), (0, 0)))
    vemb_p = jnp.pad(v_emb, ((0, 408 - (2 * NC + 1)), (0, 0)))
    mkT_p = jnp.pad(Mk, ((0, MP - MM), (0, 0))).T
    mv0_p = jnp.pad(Mv0, ((0, MP - MM), (0, 0)))

    out = pl.pallas_call(
        _main_body,
        grid=(2,),
        in_specs=[
            pl.BlockSpec((1, NROWS + BP, 1), lambda p: (p, 0, 0)),
            pl.BlockSpec((1, NROWS, 1), lambda p: (p, 0, 0)),
            pl.BlockSpec((1, NROWS, 1), lambda p: (p, 0, 0)),
            pl.BlockSpec((NS, D), lambda p: (0, 0)),
            pl.BlockSpec((208, D), lambda p: (0, 0)),
            pl.BlockSpec((408, D), lambda p: (0, 0)),
            pl.BlockSpec((D, MP), lambda p: (0, 0)),
            pl.BlockSpec((MP, D), lambda p: (0, 0)),
            pl.BlockSpec((2 * D, 1), lambda p: (0, 0)),
            pl.BlockSpec((1, 1), lambda p: (0, 0)),
            pl.BlockSpec((2 * D, 1), lambda p: (0, 0)),
            pl.BlockSpec((1, 1), lambda p: (0, 0)),
            pl.BlockSpec((D, D), lambda p: (0, 0)),
            pl.BlockSpec((1, D), lambda p: (0, 0)),
            pl.BlockSpec((D, D), lambda p: (0, 0)),
            pl.BlockSpec((1, D), lambda p: (0, 0)),
            pl.BlockSpec((2 * D, D), lambda p: (0, 0)),
            pl.BlockSpec((1, D), lambda p: (0, 0)),
            pl.BlockSpec((D, NC), lambda p: (0, 0)),
            pl.BlockSpec((1, NC), lambda p: (0, 0)),
        ],
        out_specs=pl.BlockSpec((1, NROWS, 1), lambda p: (p, 0, 0)),
        out_shape=jax.ShapeDtypeStruct((2, NROWS, 1), jnp.float32),
        scratch_shapes=[
            pltpu.VMEM((NROWS, D), jnp.float32),
            pltpu.VMEM((NROWS, MP), jnp.float32),
            pltpu.VMEM((NROWS, D), jnp.float32),
            pltpu.VMEM((NROWS, D), jnp.float32),
            pltpu.VMEM((NROWS, D), jnp.float32),
            pltpu.VMEM((BP, MP, D), jnp.float32),
        ],
        compiler_params=pltpu.CompilerParams(
            dimension_semantics=("parallel",)),
    )(skF, anF, stF, stu_emb, kemb_p, vemb_p, mkT_p, mv0_p,
      Wgk, bgk.reshape(1, 1), Wgv, bgv.reshape(1, 1),
      We, be.reshape(1, D), Wa, ba.reshape(1, D),
      Wf, bf.reshape(1, D), Wp, bp.reshape(1, NC))

    predT = out.reshape(2, LL, BP).transpose(1, 0, 2).reshape(LL, BT)
    return predT.T[:, :LL - 1]


# speedup vs baseline: 1.0576x; 1.0576x over previous
import functools

import jax
import jax.numpy as jnp
from jax.experimental import pallas as pl
from jax.experimental.pallas import tpu as pltpu

NC = 200
D = 128
MM = 50
MP = 64
NS = 4000
LL = 200
BT = 64
BP = 32
NROWS = LL * BP
CH = 800
NCHUNK = NROWS // CH
TCH = CH // BP
GBLK = 400


def _hgnn_body(g_ref, x_ref, w_ref, b_ref, o_ref, *, relu):
    xw = jnp.dot(x_ref[...], w_ref[...],
                 preferred_element_type=jnp.float32) + b_ref[...]
    y = jnp.dot(g_ref[...], xw, preferred_element_type=jnp.float32)
    o_ref[...] = jnp.maximum(y, 0.0) if relu else y


def _hgnn_layer(G, X, W, b2d, relu):
    return pl.pallas_call(
        functools.partial(_hgnn_body, relu=relu),
        grid=(NS // GBLK,),
        in_specs=[
            pl.BlockSpec((GBLK, NS), lambda i: (i, 0)),
            pl.BlockSpec((NS, D), lambda i: (0, 0)),
            pl.BlockSpec((D, D), lambda i: (0, 0)),
            pl.BlockSpec((1, D), lambda i: (0, 0)),
        ],
        out_specs=pl.BlockSpec((GBLK, D), lambda i: (i, 0)),
        out_shape=jax.ShapeDtypeStruct((NS, D), jnp.float32),
        compiler_params=pltpu.CompilerParams(
            dimension_semantics=("parallel",)),
    )(G, X, W, b2d)


def _onehot_gather(idx_col, table_val, width):
    iota = jax.lax.broadcasted_iota(jnp.int32, (idx_col.shape[0], width), 1)
    oh = jnp.where(iota == idx_col, 1.0, 0.0).astype(jnp.float32)
    return jnp.dot(oh, table_val, preferred_element_type=jnp.float32)


def _main_body(skF_ref, anF_ref, stF_ref, semb_ref, kemb_ref, vemb_ref,
               mkT_ref, mv0_ref, wgk_ref, bgk_ref, wgv_ref, bgv_ref,
               we_ref, be_ref, wa_ref, ba_ref, wf_ref, bf_ref,
               wp_ref, bp_ref,
               out_ref, k_scr, w_scr, e_scr, a_scr, r_scr, mv_scr):
    wgk_s = wgk_ref[:D, :]
    wgk_k = wgk_ref[D:, :]
    wgv_s = wgv_ref[:D, :]
    wgv_k = wgv_ref[D:, :]
    wf_r = wf_ref[:D, :]
    wf_k = wf_ref[D:, :]

    for c in range(NCHUNK):
        sk = skF_ref[0, pl.ds(c * CH, CH), :]
        an = anF_ref[0, pl.ds(c * CH, CH), :]
        st = stF_ref[0, pl.ds(c * CH, CH), :] - 1
        ax = jnp.where(an == 2, 1, an)
        xv = sk + NC * ax

        sh0 = _onehot_gather(st[:CH // 2], semb_ref[...], NS)
        sh1 = _onehot_gather(st[CH // 2:], semb_ref[...], NS)
        shc = jnp.concatenate([sh0, sh1], axis=0)

        k0 = _onehot_gather(sk, kemb_ref[...], 208)
        v0 = _onehot_gather(xv, vemb_ref[...], 408)

        gk = jax.nn.sigmoid(
            jnp.dot(shc, wgk_s, preferred_element_type=jnp.float32)
            + jnp.dot(k0, wgk_k, preferred_element_type=jnp.float32)
            + bgk_ref[...])
        kc = gk * shc + (1.0 - gk) * k0
        gv = jax.nn.sigmoid(
            jnp.dot(shc, wgv_s, preferred_element_type=jnp.float32)
            + jnp.dot(v0, wgv_k, preferred_element_type=jnp.float32)
            + bgv_ref[...])
        vc = gv * shc + (1.0 - gv) * v0

        logits = jnp.dot(kc, mkT_ref[...], preferred_element_type=jnp.float32)
        miota = jax.lax.broadcasted_iota(jnp.int32, (CH, MP), 1)
        logits = jnp.where(miota < MM, logits, -1e30)
        logits = logits - jnp.max(logits, axis=1, keepdims=True)
        ex = jnp.exp(logits)
        wc = ex / jnp.sum(ex, axis=1, keepdims=True)

        ec = jax.nn.sigmoid(
            jnp.dot(vc, we_ref[...], preferred_element_type=jnp.float32)
            + be_ref[...])
        ac = jnp.tanh(
            jnp.dot(vc, wa_ref[...], preferred_element_type=jnp.float32)
            + ba_ref[...])

        k_scr[pl.ds(c * CH, CH), :] = kc
        w_scr[pl.ds(c * CH, CH), :] = wc
        e_scr[pl.ds(c * CH, CH), :] = ec
        a_scr[pl.ds(c * CH, CH), :] = ac

    mv_scr[...] = jnp.broadcast_to(mv0_ref[...][None], (BP, MP, D))

    def step(t, carry):
        wt = w_scr[pl.ds(t * BP, BP), :]
        et = e_scr[pl.ds(t * BP, BP), :]
        at = a_scr[pl.ds(t * BP, BP), :]
        mv = mv_scr[...]
        wmv = wt[:, :, None] * mv
        r_scr[pl.ds(t * BP, BP), :] = jnp.sum(wmv, axis=1)
        mv_scr[...] = (mv - et[:, None, :] * wmv
                       + wt[:, :, None] * at[:, None, :])
        return carry

    jax.lax.fori_loop(0, LL, step, 0)

    for c in range(NCHUNK):
        rc = r_scr[pl.ds(c * CH, CH), :]
        kc = k_scr[pl.ds(c * CH, CH), :]
        fc = jnp.tanh(
            jnp.dot(rc, wf_r, preferred_element_type=jnp.float32)
            + jnp.dot(kc, wf_k, preferred_element_type=jnp.float32)
            + bf_ref[...])
        pc = jax.nn.sigmoid(
            jnp.dot(fc, wp_ref[...], preferred_element_type=jnp.float32)
            + bp_ref[...])
        nsk = skF_ref[0, pl.ds(c * CH + BP, CH), :]
        piota = jax.lax.broadcasted_iota(jnp.int32, (CH, NC), 1)
        pred = jnp.sum(jnp.where(piota == nsk, pc, 0.0), axis=1,
                       keepdims=True)
        out_ref[0, pl.ds(c * CH, CH), :] = pred


def kernel(student, skill, answer, G, stu_table, W1, b1, W2, b2, k_emb,
           v_emb, Mk, Mv0, Wgk, bgk, Wgv, bgv, We, be, Wa, ba, Wf, bf,
           Wp, bp):
    h = _hgnn_layer(G, stu_table, W1, b1.reshape(1, D), True)
    stu_emb = _hgnn_layer(G, h, W2, b2.reshape(1, D), False)

    def prep(x, pad_rows):
        xt = x.T.astype(jnp.int32).reshape(LL, 2, BP).transpose(1, 0, 2)
        xt = xt.reshape(2, NROWS, 1)
        return jnp.pad(xt, ((0, 0), (0, pad_rows), (0, 0)))
    skF = prep(skill, BP)
    anF = prep(answer, 0)
    stF = prep(student, 0)

    kemb_p = jnp.pad(k_emb, ((0, 208 - (NC + 1)), (0, 0)))
    vemb_p = jnp.pad(v_emb, ((0, 408 - (2 * NC + 1)), (0, 0)))
    mkT_p = jnp.pad(Mk, ((0, MP - MM), (0, 0))).T
    mv0_p = jnp.pad(Mv0, ((0, MP - MM), (0, 0)))

    out = pl.pallas_call(
        _main_body,
        grid=(2,),
        in_specs=[
            pl.BlockSpec((1, NROWS + BP, 1), lambda p: (p, 0, 0)),
            pl.BlockSpec((1, NROWS, 1), lambda p: (p, 0, 0)),
            pl.BlockSpec((1, NROWS, 1), lambda p: (p, 0, 0)),
            pl.BlockSpec((NS, D), lambda p: (0, 0)),
            pl.BlockSpec((208, D), lambda p: (0, 0)),
            pl.BlockSpec((408, D), lambda p: (0, 0)),
            pl.BlockSpec((D, MP), lambda p: (0, 0)),
            pl.BlockSpec((MP, D), lambda p: (0, 0)),
            pl.BlockSpec((2 * D, 1), lambda p: (0, 0)),
            pl.BlockSpec((1, 1), lambda p: (0, 0)),
            pl.BlockSpec((2 * D, 1), lambda p: (0, 0)),
            pl.BlockSpec((1, 1), lambda p: (0, 0)),
            pl.BlockSpec((D, D), lambda p: (0, 0)),
            pl.BlockSpec((1, D), lambda p: (0, 0)),
            pl.BlockSpec((D, D), lambda p: (0, 0)),
            pl.BlockSpec((1, D), lambda p: (0, 0)),
            pl.BlockSpec((2 * D, D), lambda p: (0, 0)),
            pl.BlockSpec((1, D), lambda p: (0, 0)),
            pl.BlockSpec((D, NC), lambda p: (0, 0)),
            pl.BlockSpec((1, NC), lambda p: (0, 0)),
        ],
        out_specs=pl.BlockSpec((1, NROWS, 1), lambda p: (p, 0, 0)),
        out_shape=jax.ShapeDtypeStruct((2, NROWS, 1), jnp.float32),
        scratch_shapes=[
            pltpu.VMEM((NROWS, D), jnp.float32),
            pltpu.VMEM((NROWS, MP), jnp.float32),
            pltpu.VMEM((NROWS, D), jnp.float32),
            pltpu.VMEM((NROWS, D), jnp.float32),
            pltpu.VMEM((NROWS, D), jnp.float32),
            pltpu.VMEM((BP, MP, D), jnp.float32),
        ],
        compiler_params=pltpu.CompilerParams(
            dimension_semantics=("parallel",)),
    )(skF, anF, stF, stu_emb, kemb_p, vemb_p, mkT_p, mv0_p,
      Wgk, bgk.reshape(1, 1), Wgv, bgv.reshape(1, 1),
      We, be.reshape(1, D), Wa, ba.reshape(1, D),
      Wf, bf.reshape(1, D), Wp, bp.reshape(1, NC))

    predT = out.reshape(2, LL, BP).transpose(1, 0, 2).reshape(LL, BT)
    return predT.T[:, :LL - 1]


# speedup vs baseline: 1.1096x; 1.0491x over previous
import functools

import jax
import jax.numpy as jnp
from jax.experimental import pallas as pl
from jax.experimental.pallas import tpu as pltpu

NC = 200
D = 128
MM = 50
MP = 64
NS = 4000
LL = 200
BT = 64
BP = 32
NROWS = LL * BP
CH = 800
NCHUNK = NROWS // CH
TCH = CH // BP
GBLK = 400


def _hgnn_body(g_ref, x_ref, w_ref, b_ref, o_ref, *, relu):
    xw = jnp.dot(x_ref[...], w_ref[...],
                 preferred_element_type=jnp.float32) + b_ref[...]
    y = jnp.dot(g_ref[...], xw, preferred_element_type=jnp.float32)
    o_ref[...] = jnp.maximum(y, 0.0) if relu else y


def _hgnn_layer(G, X, W, b2d, relu):
    return pl.pallas_call(
        functools.partial(_hgnn_body, relu=relu),
        grid=(NS // GBLK,),
        in_specs=[
            pl.BlockSpec((GBLK, NS), lambda i: (i, 0)),
            pl.BlockSpec((NS, D), lambda i: (0, 0)),
            pl.BlockSpec((D, D), lambda i: (0, 0)),
            pl.BlockSpec((1, D), lambda i: (0, 0)),
        ],
        out_specs=pl.BlockSpec((GBLK, D), lambda i: (i, 0)),
        out_shape=jax.ShapeDtypeStruct((NS, D), jnp.float32),
        compiler_params=pltpu.CompilerParams(
            dimension_semantics=("parallel",)),
    )(G, X, W, b2d)


def _onehot_gather(idx_col, table_val, width):
    iota = jax.lax.broadcasted_iota(jnp.int32, (idx_col.shape[0], width), 1)
    oh = jnp.where(iota == idx_col, 1.0, 0.0).astype(jnp.float32)
    return jnp.dot(oh, table_val, preferred_element_type=jnp.float32)


def _main_body(skF_ref, anF_ref, stF_ref, semb_ref, kemb_ref, vemb_ref,
               mkT_ref, mv0_ref, wgk_ref, bgk_ref, wgv_ref, bgv_ref,
               we_ref, be_ref, wa_ref, ba_ref, wf_ref, bf_ref,
               wp_ref, bp_ref,
               out_ref, k_scr, w_scr, e_scr, a_scr, r_scr, mv_scr):
    wgk_s = wgk_ref[:D, :]
    wgk_k = wgk_ref[D:, :]
    wgv_s = wgv_ref[:D, :]
    wgv_k = wgv_ref[D:, :]
    wf_r = wf_ref[:D, :]
    wf_k = wf_ref[D:, :]

    for c in range(NCHUNK):
        sk = skF_ref[0, pl.ds(c * CH, CH), :]
        an = anF_ref[0, pl.ds(c * CH, CH), :]
        st = stF_ref[0, pl.ds(c * CH, CH), :] - 1
        ax = jnp.where(an == 2, 1, an)
        xv = sk + NC * ax

        sh0 = _onehot_gather(st[:CH // 2], semb_ref[...], NS)
        sh1 = _onehot_gather(st[CH // 2:], semb_ref[...], NS)
        shc = jnp.concatenate([sh0, sh1], axis=0)

        k0 = _onehot_gather(sk, kemb_ref[...], 208)
        v0 = _onehot_gather(xv, vemb_ref[...], 408)

        gk = jax.nn.sigmoid(
            jnp.dot(shc, wgk_s, preferred_element_type=jnp.float32)
            + jnp.dot(k0, wgk_k, preferred_element_type=jnp.float32)
            + bgk_ref[...])
        kc = gk * shc + (1.0 - gk) * k0
        gv = jax.nn.sigmoid(
            jnp.dot(shc, wgv_s, preferred_element_type=jnp.float32)
            + jnp.dot(v0, wgv_k, preferred_element_type=jnp.float32)
            + bgv_ref[...])
        vc = gv * shc + (1.0 - gv) * v0

        logits = jnp.dot(kc, mkT_ref[...], preferred_element_type=jnp.float32)
        miota = jax.lax.broadcasted_iota(jnp.int32, (CH, MP), 1)
        logits = jnp.where(miota < MM, logits, -1e30)
        logits = logits - jnp.max(logits, axis=1, keepdims=True)
        ex = jnp.exp(logits)
        wc = ex / jnp.sum(ex, axis=1, keepdims=True)

        ec = jax.nn.sigmoid(
            jnp.dot(vc, we_ref[...], preferred_element_type=jnp.float32)
            + be_ref[...])
        ac = jnp.tanh(
            jnp.dot(vc, wa_ref[...], preferred_element_type=jnp.float32)
            + ba_ref[...])

        k_scr[pl.ds(c * CH, CH), :] = kc
        w_scr[pl.ds(c * CH, CH), :] = wc
        e_scr[pl.ds(c * CH, CH), :] = ec
        a_scr[pl.ds(c * CH, CH), :] = ac

    mv_scr[...] = jnp.broadcast_to(mv0_ref[...][None], (BP, MP, D))

    def step(t, carry):
        wt = w_scr[pl.ds(t * BP, BP), :]
        et = e_scr[pl.ds(t * BP, BP), :]
        at = a_scr[pl.ds(t * BP, BP), :]
        mv = mv_scr[...]
        wmv = wt[:, :, None] * mv
        r_scr[pl.ds(t * BP, BP), :] = jnp.sum(wmv, axis=1)
        mv_scr[...] = (mv - et[:, None, :] * wmv
                       + wt[:, :, None] * at[:, None, :])
        return carry

    def step2(i, carry):
        step(2 * i, carry)
        step(2 * i + 1, carry)
        return carry

    jax.lax.fori_loop(0, LL // 2, step2, 0)

    for c in range(NCHUNK):
        rc = r_scr[pl.ds(c * CH, CH), :]
        kc = k_scr[pl.ds(c * CH, CH), :]
        fc = jnp.tanh(
            jnp.dot(rc, wf_r, preferred_element_type=jnp.float32)
            + jnp.dot(kc, wf_k, preferred_element_type=jnp.float32)
            + bf_ref[...])
        pc = jax.nn.sigmoid(
            jnp.dot(fc, wp_ref[...], preferred_element_type=jnp.float32)
            + bp_ref[...])
        nsk = skF_ref[0, pl.ds(c * CH + BP, CH), :]
        piota = jax.lax.broadcasted_iota(jnp.int32, (CH, NC), 1)
        pred = jnp.sum(jnp.where(piota == nsk, pc, 0.0), axis=1,
                       keepdims=True)
        out_ref[0, pl.ds(c * CH, CH), :] = pred


def kernel(student, skill, answer, G, stu_table, W1, b1, W2, b2, k_emb,
           v_emb, Mk, Mv0, Wgk, bgk, Wgv, bgv, We, be, Wa, ba, Wf, bf,
           Wp, bp):
    h = _hgnn_layer(G, stu_table, W1, b1.reshape(1, D), True)
    stu_emb = _hgnn_layer(G, h, W2, b2.reshape(1, D), False)

    def prep(x, pad_rows):
        xt = x.T.astype(jnp.int32).reshape(LL, 2, BP).transpose(1, 0, 2)
        xt = xt.reshape(2, NROWS, 1)
        return jnp.pad(xt, ((0, 0), (0, pad_rows), (0, 0)))
    skF = prep(skill, BP)
    anF = prep(answer, 0)
    stF = prep(student, 0)

    kemb_p = jnp.pad(k_emb, ((0, 208 - (NC + 1)), (0, 0)))
    vemb_p = jnp.pad(v_emb, ((0, 408 - (2 * NC + 1)), (0, 0)))
    mkT_p = jnp.pad(Mk, ((0, MP - MM), (0, 0))).T
    mv0_p = jnp.pad(Mv0, ((0, MP - MM), (0, 0)))

    out = pl.pallas_call(
        _main_body,
        grid=(2,),
        in_specs=[
            pl.BlockSpec((1, NROWS + BP, 1), lambda p: (p, 0, 0)),
            pl.BlockSpec((1, NROWS, 1), lambda p: (p, 0, 0)),
            pl.BlockSpec((1, NROWS, 1), lambda p: (p, 0, 0)),
            pl.BlockSpec((NS, D), lambda p: (0, 0)),
            pl.BlockSpec((208, D), lambda p: (0, 0)),
            pl.BlockSpec((408, D), lambda p: (0, 0)),
            pl.BlockSpec((D, MP), lambda p: (0, 0)),
            pl.BlockSpec((MP, D), lambda p: (0, 0)),
            pl.BlockSpec((2 * D, 1), lambda p: (0, 0)),
            pl.BlockSpec((1, 1), lambda p: (0, 0)),
            pl.BlockSpec((2 * D, 1), lambda p: (0, 0)),
            pl.BlockSpec((1, 1), lambda p: (0, 0)),
            pl.BlockSpec((D, D), lambda p: (0, 0)),
            pl.BlockSpec((1, D), lambda p: (0, 0)),
            pl.BlockSpec((D, D), lambda p: (0, 0)),
            pl.BlockSpec((1, D), lambda p: (0, 0)),
            pl.BlockSpec((2 * D, D), lambda p: (0, 0)),
            pl.BlockSpec((1, D), lambda p: (0, 0)),
            pl.BlockSpec((D, NC), lambda p: (0, 0)),
            pl.BlockSpec((1, NC), lambda p: (0, 0)),
        ],
        out_specs=pl.BlockSpec((1, NROWS, 1), lambda p: (p, 0, 0)),
        out_shape=jax.ShapeDtypeStruct((2, NROWS, 1), jnp.float32),
        scratch_shapes=[
            pltpu.VMEM((NROWS, D), jnp.float32),
            pltpu.VMEM((NROWS, MP), jnp.float32),
            pltpu.VMEM((NROWS, D), jnp.float32),
            pltpu.VMEM((NROWS, D), jnp.float32),
            pltpu.VMEM((NROWS, D), jnp.float32),
            pltpu.VMEM((BP, MP, D), jnp.float32),
        ],
        compiler_params=pltpu.CompilerParams(
            dimension_semantics=("parallel",)),
    )(skF, anF, stF, stu_emb, kemb_p, vemb_p, mkT_p, mv0_p,
      Wgk, bgk.reshape(1, 1), Wgv, bgv.reshape(1, 1),
      We, be.reshape(1, D), Wa, ba.reshape(1, D),
      Wf, bf.reshape(1, D), Wp, bp.reshape(1, NC))

    predT = out.reshape(2, LL, BP).transpose(1, 0, 2).reshape(LL, BT)
    return predT.T[:, :LL - 1]


# speedup vs baseline: 1.1395x; 1.0270x over previous
import functools

import jax
import jax.numpy as jnp
from jax.experimental import pallas as pl
from jax.experimental.pallas import tpu as pltpu

NC = 200
D = 128
MM = 50
MP = 64
NS = 4000
LL = 200
BT = 64
BP = 32
NROWS = LL * BP
CH = 800
NCHUNK = NROWS // CH
TCH = CH // BP
GBLK = 400


def _hgnn_body(g_ref, x_ref, w_ref, b_ref, o_ref, *, relu):
    xw = jnp.dot(x_ref[...], w_ref[...],
                 preferred_element_type=jnp.float32) + b_ref[...]
    y = jnp.dot(g_ref[...], xw, preferred_element_type=jnp.float32)
    o_ref[...] = jnp.maximum(y, 0.0) if relu else y


def _hgnn_layer(G, X, W, b2d, relu):
    return pl.pallas_call(
        functools.partial(_hgnn_body, relu=relu),
        grid=(NS // GBLK,),
        in_specs=[
            pl.BlockSpec((GBLK, NS), lambda i: (i, 0)),
            pl.BlockSpec((NS, D), lambda i: (0, 0)),
            pl.BlockSpec((D, D), lambda i: (0, 0)),
            pl.BlockSpec((1, D), lambda i: (0, 0)),
        ],
        out_specs=pl.BlockSpec((GBLK, D), lambda i: (i, 0)),
        out_shape=jax.ShapeDtypeStruct((NS, D), jnp.float32),
        compiler_params=pltpu.CompilerParams(
            dimension_semantics=("parallel",)),
    )(G, X, W, b2d)


def _onehot_gather(idx_col, table_val, width):
    iota = jax.lax.broadcasted_iota(jnp.int32, (idx_col.shape[0], width), 1)
    oh = jnp.where(iota == idx_col, 1.0, 0.0).astype(jnp.float32)
    return jnp.dot(oh, table_val, preferred_element_type=jnp.float32)


def _main_body(skF_ref, anF_ref, stF_ref, semb_ref, kemb_ref, vemb_ref,
               mkT_ref, mv0_ref, wgk_ref, bgk_ref, wgv_ref, bgv_ref,
               we_ref, be_ref, wa_ref, ba_ref, wf_ref, bf_ref,
               wp_ref, bp_ref,
               out_ref, k_scr, w_scr, e_scr, a_scr, r_scr, mv_scr):
    wgk_s = wgk_ref[:D, :]
    wgk_k = wgk_ref[D:, :]
    wgv_s = wgv_ref[:D, :]
    wgv_k = wgv_ref[D:, :]
    wf_r = wf_ref[:D, :]
    wf_k = wf_ref[D:, :]

    for c in range(NCHUNK):
        sk = skF_ref[0, pl.ds(c * CH, CH), :]
        an = anF_ref[0, pl.ds(c * CH, CH), :]
        st = stF_ref[0, pl.ds(c * CH, CH), :] - 1
        ax = jnp.where(an == 2, 1, an)
        xv = sk + NC * ax

        sh0 = _onehot_gather(st[:CH // 2], semb_ref[...], NS)
        sh1 = _onehot_gather(st[CH // 2:], semb_ref[...], NS)
        shc = jnp.concatenate([sh0, sh1], axis=0)

        k0 = _onehot_gather(sk, kemb_ref[...], 208)
        v0 = _onehot_gather(xv, vemb_ref[...], 408)

        gk = jax.nn.sigmoid(
            jnp.dot(shc, wgk_s, preferred_element_type=jnp.float32)
            + jnp.dot(k0, wgk_k, preferred_element_type=jnp.float32)
            + bgk_ref[...])
        kc = gk * shc + (1.0 - gk) * k0
        gv = jax.nn.sigmoid(
            jnp.dot(shc, wgv_s, preferred_element_type=jnp.float32)
            + jnp.dot(v0, wgv_k, preferred_element_type=jnp.float32)
            + bgv_ref[...])
        vc = gv * shc + (1.0 - gv) * v0

        logits = jnp.dot(kc, mkT_ref[...], preferred_element_type=jnp.float32)
        miota = jax.lax.broadcasted_iota(jnp.int32, (CH, MP), 1)
        logits = jnp.where(miota < MM, logits, -1e30)
        logits = logits - jnp.max(logits, axis=1, keepdims=True)
        ex = jnp.exp(logits)
        wc = ex / jnp.sum(ex, axis=1, keepdims=True)

        ec = jax.nn.sigmoid(
            jnp.dot(vc, we_ref[...], preferred_element_type=jnp.float32)
            + be_ref[...])
        ac = jnp.tanh(
            jnp.dot(vc, wa_ref[...], preferred_element_type=jnp.float32)
            + ba_ref[...])

        k_scr[pl.ds(c * CH, CH), :] = kc
        w_scr[pl.ds(c * CH, CH), :] = wc
        e_scr[pl.ds(c * CH, CH), :] = ec
        a_scr[pl.ds(c * CH, CH), :] = ac

    mv_scr[...] = jnp.broadcast_to(mv0_ref[...][None], (BP, MP, D))

    def step(t, carry):
        wt = w_scr[pl.ds(t * BP, BP), :]
        et = e_scr[pl.ds(t * BP, BP), :]
        at = a_scr[pl.ds(t * BP, BP), :]
        mv = mv_scr[...]
        wmv = wt[:, :, None] * mv
        r_scr[pl.ds(t * BP, BP), :] = jnp.sum(wmv, axis=1)
        mv_scr[...] = (mv - et[:, None, :] * wmv
                       + wt[:, :, None] * at[:, None, :])
        return carry

    def step4(i, carry):
        for u in range(4):
            step(4 * i + u, carry)
        return carry

    jax.lax.fori_loop(0, LL // 4, step4, 0)

    for c in range(NCHUNK):
        rc = r_scr[pl.ds(c * CH, CH), :]
        kc = k_scr[pl.ds(c * CH, CH), :]
        fc = jnp.tanh(
            jnp.dot(rc, wf_r, preferred_element_type=jnp.float32)
            + jnp.dot(kc, wf_k, preferred_element_type=jnp.float32)
            + bf_ref[...])
        pc = jax.nn.sigmoid(
            jnp.dot(fc, wp_ref[...], preferred_element_type=jnp.float32)
            + bp_ref[...])
        nsk = skF_ref[0, pl.ds(c * CH + BP, CH), :]
        piota = jax.lax.broadcasted_iota(jnp.int32, (CH, NC), 1)
        pred = jnp.sum(jnp.where(piota == nsk, pc, 0.0), axis=1,
                       keepdims=True)
        out_ref[0, pl.ds(c * CH, CH), :] = pred


def kernel(student, skill, answer, G, stu_table, W1, b1, W2, b2, k_emb,
           v_emb, Mk, Mv0, Wgk, bgk, Wgv, bgv, We, be, Wa, ba, Wf, bf,
           Wp, bp):
    h = _hgnn_layer(G, stu_table, W1, b1.reshape(1, D), True)
    stu_emb = _hgnn_layer(G, h, W2, b2.reshape(1, D), False)

    def prep(x, pad_rows):
        xt = x.T.astype(jnp.int32).reshape(LL, 2, BP).transpose(1, 0, 2)
        xt = xt.reshape(2, NROWS, 1)
        return jnp.pad(xt, ((0, 0), (0, pad_rows), (0, 0)))
    skF = prep(skill, BP)
    anF = prep(answer, 0)
    stF = prep(student, 0)

    kemb_p = jnp.pad(k_emb, ((0, 208 - (NC + 1)), (0, 0)))
    vemb_p = jnp.pad(v_emb, ((0, 408 - (2 * NC + 1)), (0, 0)))
    mkT_p = jnp.pad(Mk, ((0, MP - MM), (0, 0))).T
    mv0_p = jnp.pad(Mv0, ((0, MP - MM), (0, 0)))

    out = pl.pallas_call(
        _main_body,
        grid=(2,),
        in_specs=[
            pl.BlockSpec((1, NROWS + BP, 1), lambda p: (p, 0, 0)),
            pl.BlockSpec((1, NROWS, 1), lambda p: (p, 0, 0)),
            pl.BlockSpec((1, NROWS, 1), lambda p: (p, 0, 0)),
            pl.BlockSpec((NS, D), lambda p: (0, 0)),
            pl.BlockSpec((208, D), lambda p: (0, 0)),
            pl.BlockSpec((408, D), lambda p: (0, 0)),
            pl.BlockSpec((D, MP), lambda p: (0, 0)),
            pl.BlockSpec((MP, D), lambda p: (0, 0)),
            pl.BlockSpec((2 * D, 1), lambda p: (0, 0)),
            pl.BlockSpec((1, 1), lambda p: (0, 0)),
            pl.BlockSpec((2 * D, 1), lambda p: (0, 0)),
            pl.BlockSpec((1, 1), lambda p: (0, 0)),
            pl.BlockSpec((D, D), lambda p: (0, 0)),
            pl.BlockSpec((1, D), lambda p: (0, 0)),
            pl.BlockSpec((D, D), lambda p: (0, 0)),
            pl.BlockSpec((1, D), lambda p: (0, 0)),
            pl.BlockSpec((2 * D, D), lambda p: (0, 0)),
            pl.BlockSpec((1, D), lambda p: (0, 0)),
            pl.BlockSpec((D, NC), lambda p: (0, 0)),
            pl.BlockSpec((1, NC), lambda p: (0, 0)),
        ],
        out_specs=pl.BlockSpec((1, NROWS, 1), lambda p: (p, 0, 0)),
        out_shape=jax.ShapeDtypeStruct((2, NROWS, 1), jnp.float32),
        scratch_shapes=[
            pltpu.VMEM((NROWS, D), jnp.float32),
            pltpu.VMEM((NROWS, MP), jnp.float32),
            pltpu.VMEM((NROWS, D), jnp.float32),
            pltpu.VMEM((NROWS, D), jnp.float32),
            pltpu.VMEM((NROWS, D), jnp.float32),
            pltpu.VMEM((BP, MP, D), jnp.float32),
        ],
        compiler_params=pltpu.CompilerParams(
            dimension_semantics=("parallel",)),
    )(skF, anF, stF, stu_emb, kemb_p, vemb_p, mkT_p, mv0_p,
      Wgk, bgk.reshape(1, 1), Wgv, bgv.reshape(1, 1),
      We, be.reshape(1, D), Wa, ba.reshape(1, D),
      Wf, bf.reshape(1, D), Wp, bp.reshape(1, NC))

    predT = out.reshape(2, LL, BP).transpose(1, 0, 2).reshape(LL, BT)
    return predT.T[:, :LL - 1]
